# Initial kernel scaffold; baseline (speedup 1.0000x reference)
import jax
import jax.numpy as jnp
from jax import lax
from jax.experimental import pallas as pl
from jax.experimental.pallas import tpu as pltpu
from jax.experimental.pallas import tpu_sc as plsc

N = 50000
E = 800000
F_IN = 128
H = 64
OUT = 64
G = 512

NC = 2
NS = 16
LANES = 16

NH = 25088
NPAD = NC * NH
DUMMY = 1024
NLOC = NH + DUMMY
ZROWS = 204

K = 128
ROWS_PT = 392
TOT_ROWS = NS * ROWS_PT
EPAD = TOT_ROWS * K
SENTINEL = 1 << 28

_mesh = plsc.VectorSubcoreMesh(core_axis_name="c", subcore_axis_name="s")


def _make_agg(want_cnt: bool):
    out_type = [jax.ShapeDtypeStruct((NPAD, H), jnp.float32)]
    if want_cnt:
        out_type.append(jax.ShapeDtypeStruct((NPAD,), jnp.float32))

    scratch = [
        pltpu.VMEM((2, 2, K), jnp.int32),
        pltpu.VMEM((2, K), jnp.int32),
        pltpu.VMEM((2, K, H), jnp.float32),
        pltpu.VMEM((K,), jnp.float32),
        pltpu.VMEM((ZROWS, H), jnp.float32),
        pltpu.VMEM((NLOC // NS,), jnp.float32),
        pltpu.VMEM_SHARED((NLOC, H), jnp.float32),
        pltpu.VMEM_SHARED((NLOC,), jnp.float32),
        pltpu.SemaphoreType.DMA,
        pltpu.SemaphoreType.DMA,
        pltpu.SemaphoreType.DMA,
        pltpu.SemaphoreType.DMA,
        pltpu.SemaphoreType.DMA,
        pltpu.SemaphoreType.DMA,
    ]

    def body(p_hbm, e2_hbm, *refs):
        if want_cnt:
            out_hbm, cnt_hbm = refs[0], refs[1]
            refs = refs[2:]
        else:
            out_hbm = refs[0]
            cnt_hbm = None
            refs = refs[1:]
        (idxb, mapb, rows, ones, zblk, zcnt,
         agg_sh, cnt_sh, si0, si1, sg0, sg1, ss0, ss1) = refs
        sem_i = (si0, si1)
        sem_g = (sg0, sg1)
        sem_s = (ss0, ss1)

        c = lax.axis_index("c")
        s = lax.axis_index("s")
        base = c * NH
        lane = lax.iota(jnp.int32, LANES)

        def zb_body(i, _):
            zblk[i // (H // LANES),
                 pl.ds((i % (H // LANES)) * LANES, LANES)] = jnp.zeros(
                     (LANES,), jnp.float32)
            return 0
        lax.fori_loop(0, ZROWS * H // LANES, zb_body, 0)

        def zc_body(i, _):
            zcnt[pl.ds(i * LANES, LANES)] = jnp.zeros((LANES,), jnp.float32)
            return 0
        lax.fori_loop(0, (NLOC // NS) // LANES, zc_body, 0)

        for i in range(K // LANES):
            ones[pl.ds(i * LANES, LANES)] = jnp.ones((LANES,), jnp.float32)

        zoff = s * (NLOC // NS)
        for k in range(NLOC // NS // ZROWS):
            pltpu.sync_copy(zblk, agg_sh.at[pl.ds(zoff + k * ZROWS, ZROWS)])
        pltpu.sync_copy(zcnt, cnt_sh.at[pl.ds(zoff, NLOC // NS)])
        plsc.subcore_barrier()

        row0 = s * ROWS_PT

        def compute_map(p, b):
            for i in range(K // LANES):
                v = idxb[p, 1, pl.ds(i * LANES, LANES)]
                loc = v - base
                ok = (loc >= 0) & (loc < NH)
                dmy = NH + ((b * 37 + s * 131 + i * LANES + lane) & (DUMMY - 1))
                mapb[p, pl.ds(i * LANES, LANES)] = jnp.where(ok, loc, dmy)

        def drain_scatter(p):
            pltpu.make_async_copy(rows.at[p], agg_sh.at[mapb.at[p]],
                                  sem_s[p]).wait()
            if want_cnt:
                pltpu.make_async_copy(ones, cnt_sh.at[mapb.at[p]],
                                      sem_s[p]).wait()

        pltpu.async_copy(e2_hbm.at[row0], idxb.at[0], sem_i[0])

        def outer(g, _):
            for j in range(2):
                p = j
                b = 2 * g + j

                @pl.when(g >= 1)
                def _():
                    drain_scatter(p)

                pltpu.make_async_copy(e2_hbm.at[row0], idxb.at[p],
                                      sem_i[p]).wait()
                pltpu.async_copy(p_hbm.at[idxb.at[p, 0]], rows.at[p],
                                 sem_g[p])
                if j == 0:
                    pltpu.async_copy(e2_hbm.at[row0 + b + 1], idxb.at[1 - p],
                                     sem_i[1 - p])
                else:
                    @pl.when(g < ROWS_PT // 2 - 1)
                    def _():
                        pltpu.async_copy(e2_hbm.at[row0 + b + 1],
                                         idxb.at[1 - p], sem_i[1 - p])
                compute_map(p, b)
                pltpu.make_async_copy(p_hbm.at[idxb.at[p, 0]], rows.at[p],
                                      sem_g[p]).wait()
                pltpu.async_copy(rows.at[p], agg_sh.at[mapb.at[p]], sem_s[p],
                                 add=True)
                if want_cnt:
                    pltpu.async_copy(ones, cnt_sh.at[mapb.at[p]], sem_s[p],
                                     add=True)
            return 0

        lax.fori_loop(0, ROWS_PT // 2, outer, 0)
        drain_scatter(0)
        drain_scatter(1)
        plsc.subcore_barrier()

        o = s * (NH // NS)
        pltpu.sync_copy(agg_sh.at[pl.ds(o, NH // NS)],
                        out_hbm.at[pl.ds(base + o, NH // NS)])
        if want_cnt:
            pltpu.sync_copy(cnt_sh.at[pl.ds(o, NH // NS)],
                            cnt_hbm.at[pl.ds(base + o, NH // NS)])

    return pl.kernel(body, out_type=out_type, mesh=_mesh,
                     scratch_types=scratch)


_agg_cnt = _make_agg(True)
_agg = _make_agg(False)


def _mm2(x, Wl, Wr, b, fin):
    BN = 1000

    def body(x_ref, wl_ref, wr_ref, b_ref, p_ref, r_ref):
        xb = x_ref[...]
        p_ref[...] = jnp.dot(xb, wl_ref[...], preferred_element_type=jnp.float32)
        r_ref[...] = (jnp.dot(xb, wr_ref[...], preferred_element_type=jnp.float32)
                      + b_ref[...])

    return pl.pallas_call(
        body,
        grid=(N // BN,),
        in_specs=[
            pl.BlockSpec((BN, fin), lambda i: (i, 0)),
            pl.BlockSpec((fin, H), lambda i: (0, 0)),
            pl.BlockSpec((fin, H), lambda i: (0, 0)),
            pl.BlockSpec((1, H), lambda i: (0, 0)),
        ],
        out_specs=[pl.BlockSpec((BN, H), lambda i: (i, 0)),
                   pl.BlockSpec((BN, H), lambda i: (i, 0))],
        out_shape=[jax.ShapeDtypeStruct((N, H), jnp.float32)] * 2,
    )(x, Wl, Wr, b.reshape(1, H))


def _layer_mid(agg, cnt, r1, W2l, W2r, b2):
    BN = 1000

    def body(a_ref, c_ref, r_ref, wl_ref, wr_ref, b_ref, p_ref, rr_ref):
        inv = 1.0 / jnp.maximum(c_ref[...], 1.0)
        h = jnp.maximum(a_ref[...] * inv + r_ref[...], 0.0)
        p_ref[...] = jnp.dot(h, wl_ref[...], preferred_element_type=jnp.float32)
        rr_ref[...] = (jnp.dot(h, wr_ref[...], preferred_element_type=jnp.float32)
                       + b_ref[...])

    return pl.pallas_call(
        body,
        grid=(N // BN,),
        in_specs=[
            pl.BlockSpec((BN, H), lambda i: (i, 0)),
            pl.BlockSpec((BN, 1), lambda i: (i, 0)),
            pl.BlockSpec((BN, H), lambda i: (i, 0)),
            pl.BlockSpec((H, H), lambda i: (0, 0)),
            pl.BlockSpec((H, H), lambda i: (0, 0)),
            pl.BlockSpec((1, H), lambda i: (0, 0)),
        ],
        out_specs=[pl.BlockSpec((BN, H), lambda i: (i, 0)),
                   pl.BlockSpec((BN, H), lambda i: (i, 0))],
        out_shape=[jax.ShapeDtypeStruct((N, H), jnp.float32)] * 2,
    )(agg, cnt, r1, W2l, W2r, b2.reshape(1, H))


def _pool_fc(agg2, cnt, r2, batch_rows, Wfc, bfc):
    BN = 1000
    GRID = N // BN

    def body(a_ref, c_ref, r_ref, b_ref, wfc_ref, bfc_ref, o_ref, acc, gacc):
        i = pl.program_id(0)

        @pl.when(i == 0)
        def _():
            acc[...] = jnp.zeros_like(acc)
            gacc[...] = jnp.zeros_like(gacc)

        inv = 1.0 / jnp.maximum(c_ref[...], 1.0)
        h2 = a_ref[...] * inv + r_ref[...]
        bb = b_ref[...]
        gid = lax.broadcasted_iota(jnp.int32, (G, BN), 0)
        onehot_t = (gid == bb).astype(jnp.float32)
        acc[...] += jnp.dot(onehot_t, h2, preferred_element_type=jnp.float32)
        gacc[...] += jnp.sum(onehot_t, axis=1, keepdims=True)

        @pl.when(i == GRID - 1)
        def _():
            pooled = acc[...] / jnp.maximum(gacc[...], 1.0)
            o_ref[...] = (jnp.dot(pooled, wfc_ref[...],
                                  preferred_element_type=jnp.float32)
                          + bfc_ref[...])

    return pl.pallas_call(
        body,
        grid=(GRID,),
        in_specs=[
            pl.BlockSpec((BN, H), lambda i: (i, 0)),
            pl.BlockSpec((BN, 1), lambda i: (i, 0)),
            pl.BlockSpec((BN, H), lambda i: (i, 0)),
            pl.BlockSpec((1, BN), lambda i: (i, 0)),
            pl.BlockSpec((H, OUT), lambda i: (0, 0)),
            pl.BlockSpec((1, OUT), lambda i: (0, 0)),
        ],
        out_specs=pl.BlockSpec((G, OUT), lambda i: (0, 0)),
        out_shape=jax.ShapeDtypeStruct((G, OUT), jnp.float32),
        scratch_shapes=[pltpu.VMEM((G, H), jnp.float32),
                        pltpu.VMEM((G, 1), jnp.float32)],
    )(agg2, cnt, r2, batch_rows, Wfc, bfc.reshape(1, OUT))


def _first(res):
    return res[0] if isinstance(res, (list, tuple)) else res


def kernel(x, edge_index, batch, W1l, W1r, b1, W2l, W2r, b2, Wfc, bfc):
    src = edge_index[0]
    dst = edge_index[1]
    padlen = EPAD - E
    srcp = jnp.concatenate([src, jnp.zeros((padlen,), jnp.int32)])
    dstp = jnp.concatenate([dst, jnp.full((padlen,), SENTINEL, jnp.int32)])
    e2 = jnp.stack([srcp.reshape(TOT_ROWS, K), dstp.reshape(TOT_ROWS, K)],
                   axis=1)

    p1, r1 = _mm2(x, W1l, W1r, b1, F_IN)
    agg1, cnt = _agg_cnt(p1, e2)
    agg1 = agg1[:N]
    cnt2d = cnt[:N].reshape(N, 1)

    p2, r2 = _layer_mid(agg1, cnt2d, r1, W2l, W2r, b2)
    agg2 = _first(_agg(p2, e2))[:N]

    batch_rows = batch.reshape(N // 1000, 1000)
    return _pool_fc(agg2, cnt2d, r2, batch_rows, Wfc, bfc)


# speedup vs baseline: 6.5038x; 6.5038x over previous
import jax
import jax.numpy as jnp
from jax import lax
from jax.experimental import pallas as pl
from jax.experimental.pallas import tpu as pltpu
from jax.experimental.pallas import tpu_sc as plsc

N = 50000
E = 800000
F_IN = 128
H = 64
OUT = 64
G = 512

NC = 2
NS = 16
LANES = 16

NH = 25088
NPAD = NC * NH
DUMMY = 512
NLOC = NH + DUMMY
ZROWS = 50

K = 128
ROWS_PT = 392
TOT_ROWS = NS * ROWS_PT
EPAD = TOT_ROWS * K
SENTINEL = 1 << 28

_mesh = plsc.VectorSubcoreMesh(core_axis_name="c", subcore_axis_name="s")


def _make_agg(want_cnt: bool):
    out_type = [jax.ShapeDtypeStruct((NPAD, H), jnp.float32)]
    if want_cnt:
        out_type.append(jax.ShapeDtypeStruct((NPAD,), jnp.float32))

    scratch = [
        pltpu.VMEM((2, 2, K), jnp.int32),
        pltpu.VMEM((2, K), jnp.int32),
        pltpu.VMEM((2, K, H), jnp.float32),
        pltpu.VMEM((K,), jnp.float32),
        pltpu.VMEM((ZROWS, H), jnp.float32),
        pltpu.VMEM((NLOC // NS,), jnp.float32),
        pltpu.VMEM_SHARED((NLOC, H), jnp.float32),
        pltpu.VMEM_SHARED((NLOC,), jnp.float32),
        pltpu.SemaphoreType.DMA,
        pltpu.SemaphoreType.DMA,
        pltpu.SemaphoreType.DMA,
        pltpu.SemaphoreType.DMA,
        pltpu.SemaphoreType.DMA,
        pltpu.SemaphoreType.DMA,
    ]

    def body(p_hbm, e2_hbm, *refs):
        if want_cnt:
            out_hbm, cnt_hbm = refs[0], refs[1]
            refs = refs[2:]
        else:
            out_hbm = refs[0]
            cnt_hbm = None
            refs = refs[1:]
        (idxb, mapb, rows, ones, zblk, zcnt,
         agg_sh, cnt_sh, si0, si1, sg0, sg1, ss0, ss1) = refs
        sem_i = (si0, si1)
        sem_g = (sg0, sg1)
        sem_s = (ss0, ss1)

        c = lax.axis_index("c")
        s = lax.axis_index("s")
        base = c * NH
        lane = lax.iota(jnp.int32, LANES)

        def zb_body(i, _):
            zblk[i // (H // LANES),
                 pl.ds((i % (H // LANES)) * LANES, LANES)] = jnp.zeros(
                     (LANES,), jnp.float32)
            return 0
        lax.fori_loop(0, ZROWS * H // LANES, zb_body, 0)

        def zc_body(i, _):
            zcnt[pl.ds(i * LANES, LANES)] = jnp.zeros((LANES,), jnp.float32)
            return 0
        lax.fori_loop(0, (NLOC // NS) // LANES, zc_body, 0)

        for i in range(K // LANES):
            ones[pl.ds(i * LANES, LANES)] = jnp.ones((LANES,), jnp.float32)

        zoff = s * (NLOC // NS)
        for k in range(NLOC // NS // ZROWS):
            pltpu.sync_copy(zblk, agg_sh.at[pl.ds(zoff + k * ZROWS, ZROWS)])
        pltpu.sync_copy(zcnt, cnt_sh.at[pl.ds(zoff, NLOC // NS)])
        plsc.subcore_barrier()

        row0 = s * ROWS_PT

        def compute_map(p, b):
            for i in range(K // LANES):
                v = idxb[p, 1, pl.ds(i * LANES, LANES)]
                loc = v - base
                ok = (loc >= 0) & (loc < NH)
                dmy = NH + ((b * 37 + s * 131 + i * LANES + lane) & (DUMMY - 1))
                mapb[p, pl.ds(i * LANES, LANES)] = jnp.where(ok, loc, dmy)

        def drain_scatter(p):
            pltpu.make_async_copy(rows.at[p], agg_sh.at[mapb.at[p]],
                                  sem_s[p]).wait()
            if want_cnt:
                pltpu.make_async_copy(ones, cnt_sh.at[mapb.at[p]],
                                      sem_s[p]).wait()

        pltpu.async_copy(e2_hbm.at[row0], idxb.at[0], sem_i[0])

        def outer(g, _):
            for j in range(2):
                p = j
                b = 2 * g + j

                @pl.when(g >= 1)
                def _():
                    drain_scatter(p)

                pltpu.make_async_copy(e2_hbm.at[row0], idxb.at[p],
                                      sem_i[p]).wait()
                pltpu.async_copy(p_hbm.at[idxb.at[p, 0]], rows.at[p],
                                 sem_g[p])
                if j == 0:
                    pltpu.async_copy(e2_hbm.at[row0 + b + 1], idxb.at[1 - p],
                                     sem_i[1 - p])
                else:
                    @pl.when(g < ROWS_PT // 2 - 1)
                    def _():
                        pltpu.async_copy(e2_hbm.at[row0 + b + 1],
                                         idxb.at[1 - p], sem_i[1 - p])
                compute_map(p, b)
                pltpu.make_async_copy(p_hbm.at[idxb.at[p, 0]], rows.at[p],
                                      sem_g[p]).wait()
                pltpu.async_copy(rows.at[p], agg_sh.at[mapb.at[p]], sem_s[p],
                                 add=True)
                if want_cnt:
                    pltpu.async_copy(ones, cnt_sh.at[mapb.at[p]], sem_s[p],
                                     add=True)
            return 0

        lax.fori_loop(0, ROWS_PT // 2, outer, 0)
        drain_scatter(0)
        drain_scatter(1)
        plsc.subcore_barrier()

        o = s * (NH // NS)
        pltpu.sync_copy(agg_sh.at[pl.ds(o, NH // NS)],
                        out_hbm.at[pl.ds(base + o, NH // NS)])
        if want_cnt:
            pltpu.sync_copy(cnt_sh.at[pl.ds(o, NH // NS)],
                            cnt_hbm.at[pl.ds(base + o, NH // NS)])

    return pl.kernel(
        body, out_type=out_type, mesh=_mesh, scratch_types=scratch,
        compiler_params=pltpu.CompilerParams(use_tc_tiling_on_sc=False))


_agg_cnt = _make_agg(True)
_agg = _make_agg(False)


def _mm2(x, Wl, Wr, b, fin):
    BN = 1000

    def body(x_ref, wl_ref, wr_ref, b_ref, p_ref, r_ref):
        xb = x_ref[...]
        p_ref[...] = jnp.dot(xb, wl_ref[...], preferred_element_type=jnp.float32)
        r_ref[...] = (jnp.dot(xb, wr_ref[...], preferred_element_type=jnp.float32)
                      + b_ref[...])

    return pl.pallas_call(
        body,
        grid=(N // BN,),
        in_specs=[
            pl.BlockSpec((BN, fin), lambda i: (i, 0)),
            pl.BlockSpec((fin, H), lambda i: (0, 0)),
            pl.BlockSpec((fin, H), lambda i: (0, 0)),
            pl.BlockSpec((1, H), lambda i: (0, 0)),
        ],
        out_specs=[pl.BlockSpec((BN, H), lambda i: (i, 0)),
                   pl.BlockSpec((BN, H), lambda i: (i, 0))],
        out_shape=[jax.ShapeDtypeStruct((N, H), jnp.float32)] * 2,
    )(x, Wl, Wr, b.reshape(1, H))


def _layer_mid(agg, cnt, r1, W2l, W2r, b2):
    BN = 1000

    def body(a_ref, c_ref, r_ref, wl_ref, wr_ref, b_ref, p_ref, rr_ref):
        inv = 1.0 / jnp.maximum(c_ref[...], 1.0)
        h = jnp.maximum(a_ref[...] * inv + r_ref[...], 0.0)
        p_ref[...] = jnp.dot(h, wl_ref[...], preferred_element_type=jnp.float32)
        rr_ref[...] = (jnp.dot(h, wr_ref[...], preferred_element_type=jnp.float32)
                       + b_ref[...])

    return pl.pallas_call(
        body,
        grid=(N // BN,),
        in_specs=[
            pl.BlockSpec((BN, H), lambda i: (i, 0)),
            pl.BlockSpec((BN, 1), lambda i: (i, 0)),
            pl.BlockSpec((BN, H), lambda i: (i, 0)),
            pl.BlockSpec((H, H), lambda i: (0, 0)),
            pl.BlockSpec((H, H), lambda i: (0, 0)),
            pl.BlockSpec((1, H), lambda i: (0, 0)),
        ],
        out_specs=[pl.BlockSpec((BN, H), lambda i: (i, 0)),
                   pl.BlockSpec((BN, H), lambda i: (i, 0))],
        out_shape=[jax.ShapeDtypeStruct((N, H), jnp.float32)] * 2,
    )(agg, cnt, r1, W2l, W2r, b2.reshape(1, H))


def _pool_fc(agg2, cnt, r2, batch_rows, Wfc, bfc):
    BN = 1000
    GRID = N // BN

    def body(a_ref, c_ref, r_ref, b_ref, wfc_ref, bfc_ref, o_ref, acc, gacc):
        i = pl.program_id(0)

        @pl.when(i == 0)
        def _():
            acc[...] = jnp.zeros_like(acc)
            gacc[...] = jnp.zeros_like(gacc)

        inv = 1.0 / jnp.maximum(c_ref[...], 1.0)
        h2 = a_ref[...] * inv + r_ref[...]
        bb = b_ref[0]
        gid = lax.broadcasted_iota(jnp.int32, (G, BN), 0)
        onehot_t = (gid == bb).astype(jnp.float32)
        acc[...] += jnp.dot(onehot_t, h2, preferred_element_type=jnp.float32)
        gacc[...] += jnp.sum(onehot_t, axis=1, keepdims=True)

        @pl.when(i == GRID - 1)
        def _():
            pooled = acc[...] / jnp.maximum(gacc[...], 1.0)
            o_ref[...] = (jnp.dot(pooled, wfc_ref[...],
                                  preferred_element_type=jnp.float32)
                          + bfc_ref[...])

    return pl.pallas_call(
        body,
        grid=(GRID,),
        in_specs=[
            pl.BlockSpec((BN, H), lambda i: (i, 0)),
            pl.BlockSpec((BN, 1), lambda i: (i, 0)),
            pl.BlockSpec((BN, H), lambda i: (i, 0)),
            pl.BlockSpec((1, 1, BN), lambda i: (i, 0, 0)),
            pl.BlockSpec((H, OUT), lambda i: (0, 0)),
            pl.BlockSpec((1, OUT), lambda i: (0, 0)),
        ],
        out_specs=pl.BlockSpec((G, OUT), lambda i: (0, 0)),
        out_shape=jax.ShapeDtypeStruct((G, OUT), jnp.float32),
        scratch_shapes=[pltpu.VMEM((G, H), jnp.float32),
                        pltpu.VMEM((G, 1), jnp.float32)],
    )(agg2, cnt, r2, batch_rows, Wfc, bfc.reshape(1, OUT))


def _first(res):
    return res[0] if isinstance(res, (list, tuple)) else res


def kernel(x, edge_index, batch, W1l, W1r, b1, W2l, W2r, b2, Wfc, bfc):
    src = edge_index[0]
    dst = edge_index[1]
    padlen = EPAD - E
    srcp = jnp.concatenate([src, jnp.zeros((padlen,), jnp.int32)])
    dstp = jnp.concatenate([dst, jnp.full((padlen,), SENTINEL, jnp.int32)])
    e2 = jnp.stack([srcp.reshape(TOT_ROWS, K), dstp.reshape(TOT_ROWS, K)],
                   axis=1)

    p1, r1 = _mm2(x, W1l, W1r, b1, F_IN)
    agg1, cnt = _agg_cnt(p1, e2)
    agg1 = agg1[:N]
    cnt2d = cnt[:N].reshape(N, 1)

    p2, r2 = _layer_mid(agg1, cnt2d, r1, W2l, W2r, b2)
    agg2 = _first(_agg(p2, e2))[:N]

    batch_rows = batch.reshape(N // 1000, 1, 1000)
    return _pool_fc(agg2, cnt2d, r2, batch_rows, Wfc, bfc)
